# baseline (device time: 11844 ns/iter reference)
import jax
import jax.numpy as jnp
from jax import lax
from jax.experimental import pallas as pl
from jax.experimental.pallas import tpu as pltpu

N_DEV = 4
B, SQ, SKV, HQ_SH, DH = 2, 128, 128, 4, 64
D_MODEL = 512
D_HEADS = HQ_SH * DH
ROWS = B * SQ


def kernel(x, Wq, K_ext, V_ext, Wo):
    my = lax.axis_index("i")
    Wq_l = lax.dynamic_slice_in_dim(Wq, my * D_HEADS, D_HEADS, axis=1)
    x2 = x.reshape(ROWS, x.shape[-1])

    def body(x_ref, wq_ref, k_ref, v_ref, wo_ref, out_ref,
             gath_buf, send_sems, recv_sems):
        me = lax.axis_index("i")

        barrier = pltpu.get_barrier_semaphore()
        for off in (1, 2, 3):
            pl.semaphore_signal(
                barrier, inc=1,
                device_id=(lax.rem(me + off, N_DEV),),
                device_id_type=pl.DeviceIdType.MESH,
            )

        q = jnp.dot(x_ref[...].astype(jnp.bfloat16),
                    wq_ref[...].astype(jnp.bfloat16),
                    preferred_element_type=jnp.float32) * 0.125

        def wo_blk(j):
            return wo_ref[pl.ds(j * D_HEADS, D_HEADS), :].astype(jnp.bfloat16)

        sends = []
        for b in range(B):
            for h in range(HQ_SH):
                qbh = q[b * SQ:(b + 1) * SQ, h * DH:(h + 1) * DH]
                kbh = k_ref[b, :, h, :].astype(jnp.bfloat16)
                s = lax.dot_general(qbh.astype(jnp.bfloat16), kbh,
                                    (((1,), (1,)), ((), ())),
                                    preferred_element_type=jnp.float32)
                w = jnp.exp(s)
                denom = jnp.sum(w, axis=-1, keepdims=True)
                vbh = v_ref[b, :, h, :].astype(jnp.bfloat16)
                ctx_bh = jnp.dot(w.astype(jnp.bfloat16), vbh,
                                 preferred_element_type=jnp.float32) / denom
                gath_buf[me, b * SQ:(b + 1) * SQ, h * DH:(h + 1) * DH] = (
                    ctx_bh.astype(jnp.bfloat16))
            if b == 0:
                pl.semaphore_wait(barrier, N_DEV - 1)
            for off in (1, 2, 3):
                rdma = pltpu.make_async_remote_copy(
                    src_ref=gath_buf.at[me, pl.ds(b * SQ, SQ), :],
                    dst_ref=gath_buf.at[me, pl.ds(b * SQ, SQ), :],
                    send_sem=send_sems.at[off - 1, b],
                    recv_sem=recv_sems.at[me, b],
                    device_id=(lax.rem(me + off, N_DEV),),
                    device_id_type=pl.DeviceIdType.MESH,
                )
                rdma.start()
                sends.append(rdma)

        ctx = gath_buf[me]
        wo_me = wo_blk(me)
        acc = [jnp.dot(ctx[b * SQ:(b + 1) * SQ, :], wo_me,
                       preferred_element_type=jnp.float32) for b in range(B)]

        for b in range(B):
            for off in (1, 3, 2):
                src = lax.rem(me + off, N_DEV)
                recv = pltpu.make_async_remote_copy(
                    src_ref=gath_buf.at[src, pl.ds(b * SQ, SQ), :],
                    dst_ref=gath_buf.at[src, pl.ds(b * SQ, SQ), :],
                    send_sem=send_sems.at[off - 1, b],
                    recv_sem=recv_sems.at[src, b],
                    device_id=(src,),
                    device_id_type=pl.DeviceIdType.MESH,
                )
                recv.wait_recv()
                acc[b] = acc[b] + jnp.dot(
                    gath_buf[src, b * SQ:(b + 1) * SQ, :], wo_blk(src),
                    preferred_element_type=jnp.float32)
            out_ref[b, :, :] = acc[b]

        for rdma in sends:
            rdma.wait_send()

    return pl.pallas_call(
        body,
        out_shape=jax.ShapeDtypeStruct((B, SQ, D_MODEL), jnp.float32),
        in_specs=[pl.BlockSpec(memory_space=pltpu.VMEM)] * 5,
        out_specs=pl.BlockSpec(memory_space=pltpu.VMEM),
        scratch_shapes=[
            pltpu.VMEM((N_DEV, ROWS, D_HEADS), jnp.bfloat16),
            pltpu.SemaphoreType.DMA((N_DEV - 1, B)),
            pltpu.SemaphoreType.DMA((N_DEV, B)),
        ],
        compiler_params=pltpu.CompilerParams(collective_id=0),
    )(x2, Wq_l, K_ext, V_ext, Wo)
